# baseline (speedup 1.0000x reference)
import functools

import jax
import jax.numpy as jnp
from jax import lax
from jax.experimental import pallas as pl
from jax.experimental.pallas import tpu as pltpu
from jax.experimental.pallas import tpu_sc as plsc

B, C_IN, C_OUT, N, K = 4, 32, 32, 4096, 16
NB = 512

NC, NS = 2, 16
NW = NC * NS
BSPLIT = 1
PPW = BSPLIT * N // NW
CH = 8
NCHUNK = PPW // CH


def _wx_body(w_ref, x_ref, o_ref):
    o_ref[0] = jax.lax.dot_general(
        x_ref[0], w_ref[...], (((0,), (1,)), ((), ())),
        preferred_element_type=jnp.float32)


def _knn_body(coords_ref, idx_ref):
    b = pl.program_id(0)
    i = pl.program_id(1)
    ca = coords_ref[0]
    c = coords_ref[0, :, pl.ds(i * NB, NB)]
    inner2 = jax.lax.dot_general(
        c * -2.0, ca, (((0,), (0,)), ((), ())),
        preferred_element_type=jnp.float32)
    sq_c = jnp.sum(c * c, axis=0)
    sq_p = jnp.sum(ca * ca, axis=0)
    d = (sq_c[:, None] + inner2) + sq_p[None, :]

    H = N // 2
    inf = jnp.float32(jnp.inf)
    i_l = jax.lax.broadcasted_iota(jnp.int32, (NB, H), 1)
    i_r = i_l + H
    d_l = d[:, :H]
    d_r = d[:, H:]
    cmp = d_l <= d_r
    lo = jnp.minimum(d_l, d_r)
    hi = jnp.maximum(d_l, d_r)
    loidx = jnp.where(cmp, i_l, i_r)
    hiidx = jnp.where(cmp, i_r, i_l)
    big = jnp.int32(1 << 30)
    firsts = []
    for cnt in (4, 4, 4, 2, 1, 1):
        masked = lo
        sel_acc = None
        for e in range(cnt):
            m = jnp.min(masked, axis=1, keepdims=True)
            ft = jnp.min(jnp.where(masked == m, loidx, big), axis=1,
                         keepdims=True)
            s = loidx == ft
            firsts.append(ft)
            sel_acc = s if sel_acc is None else (sel_acc | s)
            if e + 1 < cnt:
                masked = jnp.where(s, inf, masked)
        lo = jnp.where(sel_acc, hi, lo)
        loidx = jnp.where(sel_acc, hiidx, loidx)
        hi = jnp.where(sel_acc, inf, hi)
    idx_ref[0] = jnp.concatenate(firsts, axis=1) + b * N


def _sc_body(table_hbm, idx_hbm, out_hbm, idx_v, rows_v, out_v, sem):
    wid = lax.axis_index("s") * NC + lax.axis_index("c")
    base = wid * PPW
    pltpu.sync_copy(idx_hbm.at[pl.ds(base * K, PPW * K)], idx_v)

    def chunk(g, _):
        pltpu.async_copy(
            table_hbm.at[idx_v.at[pl.ds(g * CH * K, CH * K)]],
            rows_v, sem).wait()
        for p in range(CH):
            a0 = rows_v[p * K, pl.ds(0, 16)]
            a1 = rows_v[p * K, pl.ds(16, 16)]
            for j in range(1, K):
                a0 = jnp.maximum(a0, rows_v[p * K + j, pl.ds(0, 16)])
                a1 = jnp.maximum(a1, rows_v[p * K + j, pl.ds(16, 16)])
            out_v[p, pl.ds(0, 16)] = a0
            out_v[p, pl.ds(16, 16)] = a1
        pltpu.sync_copy(out_v, out_hbm.at[pl.ds(base + g * CH, CH)])
        return ()

    lax.fori_loop(0, NCHUNK, chunk, ())


_sc_gather_max = functools.partial(
    pl.kernel,
    out_type=jax.ShapeDtypeStruct((BSPLIT * N, C_OUT), jnp.float32),
    mesh=plsc.VectorSubcoreMesh(core_axis_name="c", subcore_axis_name="s"),
    scratch_types=[
        pltpu.VMEM((PPW * K,), jnp.int32),
        pltpu.VMEM((CH * K, C_OUT), jnp.float32),
        pltpu.VMEM((CH, C_OUT), jnp.float32),
        pltpu.SemaphoreType.DMA,
    ],
    compiler_params=pltpu.CompilerParams(use_tc_tiling_on_sc=False),
)(_sc_body)


@jax.jit
def kernel(x, coords, W):
    wxt = pl.pallas_call(
        _wx_body,
        grid=(B,),
        in_specs=[
            pl.BlockSpec((C_OUT, C_IN), lambda b: (0, 0)),
            pl.BlockSpec((1, C_IN, N), lambda b: (b, 0, 0)),
        ],
        out_specs=pl.BlockSpec((1, N, C_OUT), lambda b: (b, 0, 0)),
        out_shape=jax.ShapeDtypeStruct((B, N, C_OUT), jnp.float32),
    )(W, x)
    table = wxt.reshape(B * N, C_OUT)

    sel_call = pl.pallas_call(
        _knn_body,
        grid=(BSPLIT, N // NB),
        in_specs=[pl.BlockSpec((1, 3, N), lambda b, i: (b, 0, 0))],
        out_specs=pl.BlockSpec((1, NB, K), lambda b, i: (b, i, 0)),
        out_shape=jax.ShapeDtypeStruct((BSPLIT, N, K), jnp.int32),
    )

    yts = []
    for p in range(B // BSPLIT):
        cs = jax.lax.slice_in_dim(coords, p * BSPLIT, (p + 1) * BSPLIT, axis=0)
        idx = sel_call(cs) + (p * BSPLIT) * N
        yts.append(_sc_gather_max(table, idx.reshape(BSPLIT * N * K)))
    yt = jnp.concatenate(yts, axis=0)
    return (yt.reshape(B, N, C_OUT).transpose(0, 2, 1), coords)


# speedup vs baseline: 1.0275x; 1.0275x over previous
import functools

import jax
import jax.numpy as jnp
from jax import lax
from jax.experimental import pallas as pl
from jax.experimental.pallas import tpu as pltpu
from jax.experimental.pallas import tpu_sc as plsc

B, C_IN, C_OUT, N, K = 4, 32, 32, 4096, 16
NB = 512

NC, NS = 2, 16
NW = NC * NS
BSPLIT = 2
PPW = BSPLIT * N // NW
CH = 8
NCHUNK = PPW // CH


def _wx_body(w_ref, x_ref, o_ref):
    o_ref[0] = jax.lax.dot_general(
        x_ref[0], w_ref[...], (((0,), (1,)), ((), ())),
        preferred_element_type=jnp.float32)


def _knn_body(coords_ref, idx_ref):
    b = pl.program_id(0)
    i = pl.program_id(1)
    ca = coords_ref[0]
    c = coords_ref[0, :, pl.ds(i * NB, NB)]
    inner2 = jax.lax.dot_general(
        c * -2.0, ca, (((0,), (0,)), ((), ())),
        preferred_element_type=jnp.float32)
    sq_c = jnp.sum(c * c, axis=0)
    sq_p = jnp.sum(ca * ca, axis=0)
    d = (sq_c[:, None] + inner2) + sq_p[None, :]

    H = N // 2
    inf = jnp.float32(jnp.inf)
    i_l = jax.lax.broadcasted_iota(jnp.int32, (NB, H), 1)
    i_r = i_l + H
    d_l = d[:, :H]
    d_r = d[:, H:]
    cmp = d_l <= d_r
    lo = jnp.minimum(d_l, d_r)
    hi = jnp.maximum(d_l, d_r)
    loidx = jnp.where(cmp, i_l, i_r)
    hiidx = jnp.where(cmp, i_r, i_l)
    big = jnp.int32(1 << 30)
    firsts = []
    for cnt in (4, 4, 4, 2, 1, 1):
        masked = lo
        sel_acc = None
        for e in range(cnt):
            m = jnp.min(masked, axis=1, keepdims=True)
            ft = jnp.min(jnp.where(masked == m, loidx, big), axis=1,
                         keepdims=True)
            s = loidx == ft
            firsts.append(ft)
            sel_acc = s if sel_acc is None else (sel_acc | s)
            if e + 1 < cnt:
                masked = jnp.where(s, inf, masked)
        lo = jnp.where(sel_acc, hi, lo)
        loidx = jnp.where(sel_acc, hiidx, loidx)
        hi = jnp.where(sel_acc, inf, hi)
    idx_ref[0] = jnp.concatenate(firsts, axis=1) + b * N


def _sc_body(table_hbm, idx_hbm, out_hbm, idx_v, rows_v, out_v, sem):
    wid = lax.axis_index("s") * NC + lax.axis_index("c")
    base = wid * PPW
    pltpu.sync_copy(idx_hbm.at[pl.ds(base * K, PPW * K)], idx_v)

    def chunk(g, _):
        pltpu.async_copy(
            table_hbm.at[idx_v.at[pl.ds(g * CH * K, CH * K)]],
            rows_v, sem).wait()
        for p in range(CH):
            a0 = rows_v[p * K, pl.ds(0, 16)]
            a1 = rows_v[p * K, pl.ds(16, 16)]
            for j in range(1, K):
                a0 = jnp.maximum(a0, rows_v[p * K + j, pl.ds(0, 16)])
                a1 = jnp.maximum(a1, rows_v[p * K + j, pl.ds(16, 16)])
            out_v[p, pl.ds(0, 16)] = a0
            out_v[p, pl.ds(16, 16)] = a1
        pltpu.sync_copy(out_v, out_hbm.at[pl.ds(base + g * CH, CH)])
        return ()

    lax.fori_loop(0, NCHUNK, chunk, ())


_sc_gather_max = functools.partial(
    pl.kernel,
    out_type=jax.ShapeDtypeStruct((BSPLIT * N, C_OUT), jnp.float32),
    mesh=plsc.VectorSubcoreMesh(core_axis_name="c", subcore_axis_name="s"),
    scratch_types=[
        pltpu.VMEM((PPW * K,), jnp.int32),
        pltpu.VMEM((CH * K, C_OUT), jnp.float32),
        pltpu.VMEM((CH, C_OUT), jnp.float32),
        pltpu.SemaphoreType.DMA,
    ],
    compiler_params=pltpu.CompilerParams(use_tc_tiling_on_sc=False),
)(_sc_body)


@jax.jit
def kernel(x, coords, W):
    wxt = pl.pallas_call(
        _wx_body,
        grid=(B,),
        in_specs=[
            pl.BlockSpec((C_OUT, C_IN), lambda b: (0, 0)),
            pl.BlockSpec((1, C_IN, N), lambda b: (b, 0, 0)),
        ],
        out_specs=pl.BlockSpec((1, N, C_OUT), lambda b: (b, 0, 0)),
        out_shape=jax.ShapeDtypeStruct((B, N, C_OUT), jnp.float32),
    )(W, x)
    table = wxt.reshape(B * N, C_OUT)

    sel_call = pl.pallas_call(
        _knn_body,
        grid=(BSPLIT, N // NB),
        in_specs=[pl.BlockSpec((1, 3, N), lambda b, i: (b, 0, 0))],
        out_specs=pl.BlockSpec((1, NB, K), lambda b, i: (b, i, 0)),
        out_shape=jax.ShapeDtypeStruct((BSPLIT, N, K), jnp.int32),
    )

    yts = []
    for p in range(B // BSPLIT):
        cs = jax.lax.slice_in_dim(coords, p * BSPLIT, (p + 1) * BSPLIT, axis=0)
        idx = sel_call(cs) + (p * BSPLIT) * N
        yts.append(_sc_gather_max(table, idx.reshape(BSPLIT * N * K)))
    yt = jnp.concatenate(yts, axis=0)
    return (yt.reshape(B, N, C_OUT).transpose(0, 2, 1), coords)
